# baseline (speedup 1.0000x reference)
import functools

import jax
import jax.numpy as jnp
from jax import lax
from jax.experimental import pallas as pl
from jax.experimental.pallas import tpu as pltpu
from jax.experimental.pallas import tpu_sc as plsc

D = 128
DEG = 32
FAN = DEG + 1
G = 3
GPAD = 104
NC = 2
NS = 16
NW = NC * NS


def _mm_body(x_ref, w_ref, b_ref, o_ref):
    o_ref[...] = (
        jnp.dot(x_ref[...], w_ref[0], preferred_element_type=jnp.float32)
        + b_ref[0]
    )


def _project_table(feat_table, W1, b1):
    n, d = feat_table.shape
    wt = W1.T.astype(jnp.float32)
    wstack = jnp.stack([wt[:d], wt[d:] * (1.0 / DEG)])
    bstack = jnp.stack([b1, jnp.zeros_like(b1)])[:, None, :]
    nb = 5
    bm = n // nb
    return pl.pallas_call(
        _mm_body,
        grid=(2, nb),
        in_specs=[
            pl.BlockSpec((bm, d), lambda g, i: (i, 0)),
            pl.BlockSpec((1, d, d), lambda g, i: (g, 0, 0)),
            pl.BlockSpec((1, 1, d), lambda g, i: (g, 0, 0)),
        ],
        out_specs=pl.BlockSpec((bm, d), lambda g, i: (g * nb + i, 0)),
        out_shape=jax.ShapeDtypeStruct((2 * n, d), jnp.float32),
    )(feat_table, wstack, bstack)


def _make_bag_kernel(ni, b_per_w, bpad):
    mesh = plsc.VectorSubcoreMesh(core_axis_name="c", subcore_axis_name="s")

    @functools.partial(
        pl.kernel,
        mesh=mesh,
        out_type=jax.ShapeDtypeStruct((bpad, D), jnp.float32),
        scratch_types=[
            pltpu.VMEM((ni, GPAD), jnp.int32),
            pltpu.VMEM((2, GPAD, D), jnp.float32),
            pltpu.VMEM((8 * G, D), jnp.float32),
            pltpu.SemaphoreType.DMA,
            pltpu.SemaphoreType.DMA,
        ],
    )
    def bag(t_hbm, idx_hbm, out_hbm, idx_v, rows_v, out_v, sem0, sem1):
        cid = lax.axis_index("c")
        sid = lax.axis_index("s")
        wid = sid * NC + cid
        pltpu.sync_copy(idx_hbm.at[wid], idx_v)
        pltpu.async_copy(t_hbm.at[idx_v.at[0]], rows_v.at[0], sem0)
        pltpu.async_copy(t_hbm.at[idx_v.at[1]], rows_v.at[1], sem1)

        def process(t, buf, sem):
            pltpu.make_async_copy(
                t_hbm.at[idx_v.at[t]], rows_v.at[buf], sem
            ).wait()
            slab = rows_v.at[buf]
            stage = (t % 8) * G
            for g in range(G):
                for c in range(D // 16):
                    sl = pl.ds(c * 16, 16)
                    vals = [slab[FAN * g + j, sl] for j in range(FAN)]
                    while len(vals) > 1:
                        nxt = [
                            vals[i] + vals[i + 1]
                            for i in range(0, len(vals) - 1, 2)
                        ]
                        if len(vals) % 2:
                            nxt.append(vals[-1])
                        vals = nxt
                    out_v[stage + g, sl] = jnp.maximum(vals[0], 0.0)

            @pl.when(t + 2 < ni)
            def _():
                pltpu.async_copy(t_hbm.at[idx_v.at[t + 2]], rows_v.at[buf], sem)

        def body(p, carry):
            t = 2 * p
            process(t, 0, sem0)
            process(t + 1, 1, sem1)

            @pl.when(p % 4 == 3)
            def _():
                pltpu.sync_copy(
                    out_v,
                    out_hbm.at[pl.ds(wid * b_per_w + (p // 4) * (8 * G), 8 * G)],
                )

            return carry

        lax.fori_loop(0, ni // 2, body, 0)

    return bag


def kernel(feat_table, W1, b1, nodes, neigh_index):
    n_nodes, d = feat_table.shape
    b = nodes.shape[0]
    ni = -(-b // (NW * G * 8)) * 8
    b_per_w = ni * G
    bpad = NW * b_per_w

    t_proj = _project_table(feat_table, W1, b1)

    idx = jnp.concatenate(
        [
            nodes.astype(jnp.int32)[:, None],
            neigh_index.astype(jnp.int32) + jnp.int32(n_nodes),
        ],
        axis=1,
    )
    idx = jnp.pad(idx, ((0, bpad - b), (0, 0)))
    idx = idx.reshape(bpad // G, G * FAN)
    idx = jnp.pad(idx, ((0, 0), (0, GPAD - G * FAN)))
    idx = idx.reshape(NW, ni, GPAD)

    out = _make_bag_kernel(ni, b_per_w, bpad)(t_proj, idx)
    return out[:b]


# speedup vs baseline: 6.5205x; 6.5205x over previous
import functools

import jax
import jax.numpy as jnp
from jax import lax
from jax.experimental import pallas as pl
from jax.experimental.pallas import tpu as pltpu
from jax.experimental.pallas import tpu_sc as plsc

D = 128
DEG = 32
FAN = DEG + 1
G = 3
GPAD = 104
NC = 2
NS = 16
NW = NC * NS


def _mm_body(x_ref, w_ref, b_ref, o_ref):
    o_ref[...] = (
        jnp.dot(x_ref[...], w_ref[0], preferred_element_type=jnp.float32)
        + b_ref[0]
    )


def _project_table(feat_table, W1, b1):
    n, d = feat_table.shape
    wt = W1.T.astype(jnp.float32)
    wstack = jnp.stack([wt[:d], wt[d:] * (1.0 / DEG)])
    bstack = jnp.stack([b1, jnp.zeros_like(b1)])[:, None, :]
    nb = 5
    bm = n // nb
    return pl.pallas_call(
        _mm_body,
        grid=(2, nb),
        in_specs=[
            pl.BlockSpec((bm, d), lambda g, i: (i, 0)),
            pl.BlockSpec((1, d, d), lambda g, i: (g, 0, 0)),
            pl.BlockSpec((1, 1, d), lambda g, i: (g, 0, 0)),
        ],
        out_specs=pl.BlockSpec((bm, d), lambda g, i: (g * nb + i, 0)),
        out_shape=jax.ShapeDtypeStruct((2 * n, d), jnp.float32),
    )(feat_table, wstack, bstack)


def _make_bag_kernel(ni, b_per_w, bpad):
    mesh = plsc.VectorSubcoreMesh(core_axis_name="c", subcore_axis_name="s")

    @functools.partial(
        pl.kernel,
        mesh=mesh,
        out_type=jax.ShapeDtypeStruct((bpad, D), jnp.float32),
        scratch_types=[
            pltpu.VMEM((ni, GPAD), jnp.int32),
            pltpu.VMEM((2, GPAD, D), jnp.float32),
            pltpu.VMEM((8 * G, D), jnp.float32),
            pltpu.SemaphoreType.DMA,
            pltpu.SemaphoreType.DMA,
        ],
    )
    def bag(t_hbm, idx_hbm, out_hbm, idx_v, rows_v, out_v, sem0, sem1):
        cid = lax.axis_index("c")
        sid = lax.axis_index("s")
        wid = sid * NC + cid
        pltpu.sync_copy(idx_hbm.at[wid], idx_v)
        pltpu.async_copy(t_hbm.at[idx_v.at[0]], rows_v.at[0], sem0)
        pltpu.async_copy(t_hbm.at[idx_v.at[1]], rows_v.at[1], sem1)

        def process(t, buf, sem):
            pltpu.make_async_copy(
                t_hbm.at[idx_v.at[t]], rows_v.at[buf], sem
            ).wait()
            slab = rows_v.at[buf]
            stage = (t % 8) * G
            for g in range(G):
                for c in range(D // 16):
                    sl = pl.ds(c * 16, 16)
                    vals = [slab[FAN * g + j, sl] for j in range(FAN)]
                    while len(vals) > 1:
                        nxt = [
                            vals[i] + vals[i + 1]
                            for i in range(0, len(vals) - 1, 2)
                        ]
                        if len(vals) % 2:
                            nxt.append(vals[-1])
                        vals = nxt
                    out_v[stage + g, sl] = jnp.maximum(vals[0], 0.0)

            @pl.when(t + 2 < ni)
            def _():
                pltpu.async_copy(t_hbm.at[idx_v.at[t + 2]], rows_v.at[buf], sem)

        def body(p, carry):
            t = 2 * p
            process(t, 0, sem0)
            process(t + 1, 1, sem1)

            @pl.when(p % 4 == 3)
            def _():
                pltpu.sync_copy(
                    out_v,
                    out_hbm.at[pl.ds(wid * b_per_w + (p // 4) * (8 * G), 8 * G)],
                )

            return carry

        lax.fori_loop(0, ni // 2, body, 0)

    return bag


def kernel(feat_table, W1, b1, nodes, neigh_index):
    n_nodes, d = feat_table.shape
    b = nodes.shape[0]
    ni = -(-b // (NW * G * 8)) * 8
    b_per_w = ni * G
    bpad = NW * b_per_w

    t_proj = _project_table(feat_table, W1, b1)

    idx = jnp.concatenate(
        [
            nodes.astype(jnp.int32)[:, None],
            neigh_index.astype(jnp.int32) + jnp.int32(n_nodes),
        ],
        axis=1,
    )
    nrow_pad = bpad - b
    row_fill = (
        jnp.arange(nrow_pad * FAN, dtype=jnp.int32) % jnp.int32(2 * n_nodes)
    ).reshape(nrow_pad, FAN)
    idx = jnp.concatenate([idx, row_fill], axis=0)
    idx = idx.reshape(bpad // G, G * FAN)
    ncol_pad = GPAD - G * FAN
    col_fill = (
        jnp.arange((bpad // G) * ncol_pad, dtype=jnp.int32)
        % jnp.int32(2 * n_nodes)
    ).reshape(bpad // G, ncol_pad)
    idx = jnp.concatenate([idx, col_fill], axis=1)
    idx = idx.reshape(NW, ni, GPAD)

    out = _make_bag_kernel(ni, b_per_w, bpad)(t_proj, idx)
    return out[:b]
